# baseline (speedup 1.0000x reference)
import functools

import jax
import jax.numpy as jnp
from jax.experimental import pallas as pl
from jax.experimental.pallas import tpu as pltpu
from jax.experimental.pallas import tpu_sc as plsc

_B = 16384
_EMB = 32
_NUMC = 13
_VOCABS = (1000,) * 6 + (100,) * 20
_NF = len(_VOCABS)
_NIDX = _NF * _B
_TOTV = sum(_VOCABS)

_WINDOW = 512
_TBLK = 2048


def _cats_gather(table, idx, n):
    mesh = plsc.VectorSubcoreMesh(core_axis_name="c", subcore_axis_name="s")

    @functools.partial(
        pl.kernel,
        out_type=jax.ShapeDtypeStruct((n, _EMB), jnp.float32),
        mesh=mesh,
        compiler_params=pltpu.CompilerParams(use_tc_tiling_on_sc=False),
    )
    def k(tbl_hbm, idx_hbm, out_hbm):
        def body(i_vmem, o_vmem):
            pltpu.sync_copy(tbl_hbm.at[i_vmem], o_vmem)

        pltpu.emit_pipeline(
            body,
            grid=(n // _WINDOW,),
            in_specs=[pl.BlockSpec((_WINDOW,), lambda i: (i,))],
            out_specs=[pl.BlockSpec((_WINDOW, _EMB), lambda i: (i, 0))],
            core_axis_name=("c", "s"),
            dimension_semantics=(pltpu.PARALLEL,),
        )(idx_hbm, out_hbm)

    return k(table, idx)


def _prep(nums_stacked, cats_stacked):
    def body(n_ref, c_ref, xn_ref, idx_ref):
        xn_ref[...] = n_ref[...].T
        f = jax.lax.broadcasted_iota(jnp.int32, (1, _NF), 1)
        off = jnp.where(f < 6, f * 1000, 6000 + (f - 6) * 100)
        idx_ref[...] = c_ref[...].T + off

    return pl.pallas_call(
        body,
        grid=(_B // _TBLK,),
        in_specs=[
            pl.BlockSpec((_NUMC, _TBLK), lambda j: (0, j)),
            pl.BlockSpec((_NF, _TBLK), lambda j: (0, j)),
        ],
        out_specs=[
            pl.BlockSpec((_TBLK, _NUMC), lambda j: (j, 0)),
            pl.BlockSpec((_TBLK, _NF), lambda j: (j, 0)),
        ],
        out_shape=[
            jax.ShapeDtypeStruct((_B, _NUMC), jnp.float32),
            jax.ShapeDtypeStruct((_B, _NF), jnp.int32),
        ],
    )(nums_stacked, cats_stacked)


def kernel(num_0, num_1, num_2, num_3, num_4, num_5, num_6, num_7, num_8,
           num_9, num_10, num_11, num_12,
           cat_0, cat_1, cat_2, cat_3, cat_4, cat_5, cat_6, cat_7, cat_8,
           cat_9, cat_10, cat_11, cat_12, cat_13, cat_14, cat_15, cat_16,
           cat_17, cat_18, cat_19, cat_20, cat_21, cat_22, cat_23, cat_24,
           cat_25,
           W_0, W_1, W_2, W_3, W_4, W_5, W_6, W_7, W_8, W_9, W_10, W_11,
           W_12, W_13, W_14, W_15, W_16, W_17, W_18, W_19, W_20, W_21,
           W_22, W_23, W_24, W_25):
    nums = [num_0, num_1, num_2, num_3, num_4, num_5, num_6, num_7, num_8,
            num_9, num_10, num_11, num_12]
    cats = [cat_0, cat_1, cat_2, cat_3, cat_4, cat_5, cat_6, cat_7, cat_8,
            cat_9, cat_10, cat_11, cat_12, cat_13, cat_14, cat_15, cat_16,
            cat_17, cat_18, cat_19, cat_20, cat_21, cat_22, cat_23, cat_24,
            cat_25]
    tables = [W_0, W_1, W_2, W_3, W_4, W_5, W_6, W_7, W_8, W_9, W_10, W_11,
              W_12, W_13, W_14, W_15, W_16, W_17, W_18, W_19, W_20, W_21,
              W_22, W_23, W_24, W_25]

    table = jnp.concatenate(tables, axis=0)
    x_nums, idx = _prep(jnp.stack(nums, axis=0), jnp.stack(cats, axis=0))

    half = _B // 2
    idx_flat = idx.reshape(_NIDX)
    g0 = _cats_gather(table, idx_flat[: half * _NF], _NIDX // 2)
    g1 = _cats_gather(table, idx_flat[half * _NF:], _NIDX // 2)
    x_cats = jnp.concatenate([g0.reshape(half, _NF * _EMB),
                              g1.reshape(half, _NF * _EMB)], axis=0)
    return (x_nums, x_cats)


# speedup vs baseline: 1.3662x; 1.3662x over previous
import functools

import jax
import jax.numpy as jnp
from jax import lax
from jax.experimental import pallas as pl
from jax.experimental.pallas import tpu as pltpu
from jax.experimental.pallas import tpu_sc as plsc

_B = 16384
_EMB = 32
_NUMC = 13
_VOCABS = (1000,) * 6 + (100,) * 20
_NF = len(_VOCABS)
_NIDX = _NF * _B
_TOTV = sum(_VOCABS)

_WINDOW = 1024
_TBLK = 2048


def _cats_gather(table, idx):
    mesh = plsc.VectorSubcoreMesh(core_axis_name="c", subcore_axis_name="s")

    @functools.partial(
        pl.kernel,
        out_type=jax.ShapeDtypeStruct((_NIDX, _EMB), jnp.float32),
        mesh=mesh,
        compiler_params=pltpu.CompilerParams(use_tc_tiling_on_sc=False),
        scratch_types=[pltpu.VMEM_SHARED((_TOTV, _EMB), jnp.float32)],
    )
    def k(tbl_hbm, idx_hbm, out_hbm, tbl_sp):
        sid = lax.axis_index("s")

        @pl.when(sid == 0)
        def _():
            pltpu.sync_copy(tbl_hbm, tbl_sp)

        plsc.subcore_barrier()

        def body(i_vmem, o_vmem):
            pltpu.sync_copy(tbl_sp.at[i_vmem], o_vmem)

        pltpu.emit_pipeline(
            body,
            grid=(_NIDX // _WINDOW,),
            in_specs=[pl.BlockSpec((_WINDOW,), lambda i: (i,))],
            out_specs=[pl.BlockSpec((_WINDOW, _EMB), lambda i: (i, 0))],
            core_axis_name=("c", "s"),
            dimension_semantics=(pltpu.PARALLEL,),
        )(idx_hbm, out_hbm)

    return k(table, idx)


def _prep(nums_stacked, cats_stacked):
    def body(n_ref, c_ref, xn_ref, idx_ref):
        xn_ref[...] = n_ref[...].T
        f = jax.lax.broadcasted_iota(jnp.int32, (1, _NF), 1)
        off = jnp.where(f < 6, f * 1000, 6000 + (f - 6) * 100)
        idx_ref[...] = c_ref[...].T + off

    return pl.pallas_call(
        body,
        grid=(_B // _TBLK,),
        in_specs=[
            pl.BlockSpec((_NUMC, _TBLK), lambda j: (0, j)),
            pl.BlockSpec((_NF, _TBLK), lambda j: (0, j)),
        ],
        out_specs=[
            pl.BlockSpec((_TBLK, _NUMC), lambda j: (j, 0)),
            pl.BlockSpec((_TBLK, _NF), lambda j: (j, 0)),
        ],
        out_shape=[
            jax.ShapeDtypeStruct((_B, _NUMC), jnp.float32),
            jax.ShapeDtypeStruct((_B, _NF), jnp.int32),
        ],
    )(nums_stacked, cats_stacked)


def kernel(num_0, num_1, num_2, num_3, num_4, num_5, num_6, num_7, num_8,
           num_9, num_10, num_11, num_12,
           cat_0, cat_1, cat_2, cat_3, cat_4, cat_5, cat_6, cat_7, cat_8,
           cat_9, cat_10, cat_11, cat_12, cat_13, cat_14, cat_15, cat_16,
           cat_17, cat_18, cat_19, cat_20, cat_21, cat_22, cat_23, cat_24,
           cat_25,
           W_0, W_1, W_2, W_3, W_4, W_5, W_6, W_7, W_8, W_9, W_10, W_11,
           W_12, W_13, W_14, W_15, W_16, W_17, W_18, W_19, W_20, W_21,
           W_22, W_23, W_24, W_25):
    nums = [num_0, num_1, num_2, num_3, num_4, num_5, num_6, num_7, num_8,
            num_9, num_10, num_11, num_12]
    cats = [cat_0, cat_1, cat_2, cat_3, cat_4, cat_5, cat_6, cat_7, cat_8,
            cat_9, cat_10, cat_11, cat_12, cat_13, cat_14, cat_15, cat_16,
            cat_17, cat_18, cat_19, cat_20, cat_21, cat_22, cat_23, cat_24,
            cat_25]
    tables = [W_0, W_1, W_2, W_3, W_4, W_5, W_6, W_7, W_8, W_9, W_10, W_11,
              W_12, W_13, W_14, W_15, W_16, W_17, W_18, W_19, W_20, W_21,
              W_22, W_23, W_24, W_25]

    table = jnp.concatenate(tables, axis=0)
    x_nums, idx = _prep(jnp.stack(nums, axis=0), jnp.stack(cats, axis=0))

    gathered = _cats_gather(table, idx.reshape(_NIDX))
    x_cats = gathered.reshape(_B, _NF * _EMB)
    return (x_nums, x_cats)
